# baseline (speedup 1.0000x reference)
import jax
import jax.numpy as jnp
from jax import lax
from jax.experimental import pallas as pl
from jax.experimental.pallas import tpu as pltpu
from jax.experimental.pallas import tpu_sc as plsc

N = 10000
D = 128
DE = 16
E = 320000

NC, NS = 2, 16
NW = NC * NS
NCH = 80
EPAD = NW * NCH * 128
CPT = (EPAD // 128) // NS
AGG = 10112
ROWS_PER_TILE = AGG // NS

_mesh = plsc.VectorSubcoreMesh(core_axis_name="c", subcore_axis_name="s")


NBUF = 4


def _gather_body(x_hbm, src2d_hbm, gx_hbm, idx_v, b0, b1, b2, b3, gsem, wsem):
    bufs = (b0, b1, b2, b3)
    c = lax.axis_index("c")
    s = lax.axis_index("s")
    wid = s * NC + c
    row0 = wid * NCH
    pltpu.sync_copy(src2d_hbm.at[pl.ds(row0, NCH)], idx_v)
    for b in range(NBUF):
        pltpu.async_copy(x_hbm.at[idx_v.at[b]], bufs[b], gsem.at[b])

    def outer(k, carry):
        for b in range(NBUF):
            j = k * NBUF + b
            pltpu.make_async_copy(x_hbm.at[pl.ds(0, 128)],
                                  bufs[b], gsem.at[b]).wait()
            pltpu.async_copy(bufs[b],
                             gx_hbm.at[pl.ds((row0 + j) * 128, 128)], wsem.at[b])
            pltpu.make_async_copy(bufs[b],
                                  gx_hbm.at[pl.ds(0, 128)], wsem.at[b]).wait()

            @pl.when(j + NBUF < NCH)
            def _next():
                pltpu.async_copy(x_hbm.at[idx_v.at[j + NBUF]],
                                 bufs[b], gsem.at[b])
        return carry

    lax.fori_loop(0, NCH // NBUF, outer, 0)


@jax.jit
def _sc_gather(x, src2d):
    return pl.kernel(
        _gather_body,
        out_type=jax.ShapeDtypeStruct((EPAD, D), jnp.float32),
        mesh=_mesh,
        scratch_types=[
            pltpu.VMEM((NCH, 128), jnp.int32),
            pltpu.VMEM((128, D), jnp.float32),
            pltpu.VMEM((128, D), jnp.float32),
            pltpu.VMEM((128, D), jnp.float32),
            pltpu.VMEM((128, D), jnp.float32),
            pltpu.SemaphoreType.DMA((NBUF,)),
            pltpu.SemaphoreType.DMA((NBUF,)),
        ],
    )(x, src2d)


def _scatter_body(msg_hbm, dst2d_hbm, zeros_hbm, out_hbm, idx_v, mbuf, acc):
    c = lax.axis_index("c")
    s = lax.axis_index("s")
    row0 = s * CPT
    pltpu.sync_copy(dst2d_hbm.at[pl.ds(row0, CPT)], idx_v)

    @pl.when(s == 0)
    def _zero():
        pltpu.sync_copy(zeros_hbm, acc)

    plsc.subcore_barrier()

    def body(j, carry):
        pltpu.sync_copy(msg_hbm.at[c, pl.ds((row0 + j) * 128, 128)], mbuf)
        pltpu.sync_copy(mbuf, acc.at[idx_v.at[j]], add=True)
        return carry

    lax.fori_loop(0, CPT, body, 0)
    plsc.subcore_barrier()
    pltpu.sync_copy(acc.at[pl.ds(s * ROWS_PER_TILE, ROWS_PER_TILE)],
                    out_hbm.at[c, pl.ds(s * ROWS_PER_TILE, ROWS_PER_TILE)])


@jax.jit
def _sc_scatter(msg, dst2d, zeros):
    return pl.kernel(
        _scatter_body,
        out_type=jax.ShapeDtypeStruct((NC, AGG, 128), jnp.float32),
        mesh=_mesh,
        scratch_types=[
            pltpu.VMEM((CPT, 128), jnp.int32),
            pltpu.VMEM((128, 128), jnp.float32),
            pltpu.VMEM_SHARED((AGG, 128), jnp.float32),
        ],
    )(msg, dst2d, zeros)


EB = 4096


def _edge_mlp_body(gx_ref, ea_ref, ew_ref, ebias_ref, out_ref):
    gxt = gx_ref[...]
    ea = ea_ref[...]
    for c in range(NC):
        e = jnp.dot(ea, ew_ref[c], preferred_element_type=jnp.float32)
        out_ref[c] = jnp.maximum(gxt + e + ebias_ref[c], 0.0)


@jax.jit
def _tc_edge(gx, ea_p, ew, ebias):
    return pl.pallas_call(
        _edge_mlp_body,
        grid=(EPAD // EB,),
        in_specs=[
            pl.BlockSpec((EB, D), lambda i: (i, 0)),
            pl.BlockSpec((EB, DE), lambda i: (i, 0)),
            pl.BlockSpec((NC, DE, D), lambda i: (0, 0, 0)),
            pl.BlockSpec((NC, 1, D), lambda i: (0, 0, 0)),
        ],
        out_specs=pl.BlockSpec((NC, EB, D), lambda i: (0, i, 0)),
        out_shape=jax.ShapeDtypeStruct((NC, EPAD, D), jnp.float32),
    )(gx, ea_p, ew, ebias)


RB = 2000
NSTEP = N // RB


def _affine_gelu(h, st, cs, gb):
    m = st[0:1, :] * (1.0 / N)
    v = cs[0:1, :] * (1.0 / N)
    scale = gb[0:1, :] * lax.rsqrt(v + 1e-5)
    h = h * scale + (gb[1:2, :] - m * scale)
    return h * 0.5 * (1.0 + lax.erf(h * 0.7071067811865476))


def _csq_body(h_ref, st_ref, cs_ref):
    i = pl.program_id(0)
    m = st_ref[0:1, :] * (1.0 / N)
    d = h_ref[...] - m
    val = jnp.concatenate([jnp.sum(d * d, 0, keepdims=True),
                           jnp.zeros((7, d.shape[1]), jnp.float32)], 0)

    @pl.when(i == 0)
    def _init():
        cs_ref[...] = val

    @pl.when(i > 0)
    def _acc():
        cs_ref[...] = cs_ref[...] + val


@jax.jit
def _tc_csq(h, st):
    return pl.pallas_call(
        _csq_body,
        grid=(NSTEP,),
        in_specs=[pl.BlockSpec((RB, 64), lambda i: (i, 0)),
                  pl.BlockSpec((8, 64), lambda i: (0, 0))],
        out_specs=pl.BlockSpec((8, 64), lambda i: (0, 0)),
        out_shape=jax.ShapeDtypeStruct((8, 64), jnp.float32),
    )(h, st)


def _accum_stats(i, st_ref, h):
    s = jnp.sum(h, 0, keepdims=True)
    sq = jnp.sum(h * h, 0, keepdims=True)
    val = jnp.concatenate([s, sq, jnp.zeros((6, h.shape[1]), jnp.float32)], 0)

    @pl.when(i == 0)
    def _init():
        st_ref[...] = val

    @pl.when(i > 0)
    def _acc():
        st_ref[...] = st_ref[...] + val


def _k1_body(x_ref, agg_ref, l1a_ref, l1v_ref, esc_ref, b1_ref, h1_ref, st_ref):
    i = pl.program_id(0)
    x = x_ref[...]
    ha = jnp.dot(x * esc_ref[0, 0] + agg_ref[0],
                 l1a_ref[...], preferred_element_type=jnp.float32)
    hv = jnp.dot(x * esc_ref[0, 1] + agg_ref[1],
                 l1v_ref[...], preferred_element_type=jnp.float32)
    h = jnp.concatenate([ha, hv], axis=1) + b1_ref[...]
    h1_ref[...] = h
    _accum_stats(i, st_ref, h)


def _k2_body(h1_ref, st1_ref, cs1_ref, l2_ref, gb1_ref, b2_ref, h2_ref, st_ref):
    i = pl.program_id(0)
    g = _affine_gelu(h1_ref[...], st1_ref[...], cs1_ref[...], gb1_ref[...])
    h = jnp.dot(g, l2_ref[...], preferred_element_type=jnp.float32) + b2_ref[...]
    h2_ref[...] = h
    _accum_stats(i, st_ref, h)


def _k3_body(h2_ref, st2_ref, cs2_ref, gb2_ref, obd_ref, obc_ref, w4_ref, b4_ref,
             out_ref):
    g = _affine_gelu(h2_ref[...], st2_ref[...], cs2_ref[...], gb2_ref[...])
    ho = jnp.dot(g, obd_ref[...], preferred_element_type=jnp.float32) + obc_ref[...]
    r = jnp.dot(ho, w4_ref[...], preferred_element_type=jnp.float32) + b4_ref[...]
    lane = lax.broadcasted_iota(jnp.int32, r.shape, 1)
    lp_tot = jnp.zeros((RB, 1), jnp.float32)
    ent_tot = jnp.zeros((RB, 1), jnp.float32)
    ams = []
    for lo, hi in ((0, 36), (36, 46), (46, 54)):
        msk = (lane >= lo) & (lane < hi)
        lm = jnp.where(msk, r, -3e38)
        mx = jnp.max(lm, 1, keepdims=True)
        ex = jnp.where(msk, jnp.exp(r - mx), 0.0)
        s = jnp.sum(ex, 1, keepdims=True)
        lse = jnp.log(s) + mx
        am = jnp.min(jnp.where(lm == mx, lane, 10 ** 9), 1, keepdims=True)
        ams.append((am - lo).astype(jnp.float32))
        lp_tot = lp_tot + (mx - lse)
        ent_tot = ent_tot - jnp.sum(ex * (r - lse), 1, keepdims=True) / s
    val = r[:, 64:65]
    out_ref[...] = jnp.concatenate(
        ams + [lp_tot, ent_tot, val, jnp.zeros((RB, 2), jnp.float32)], axis=1)


def _full(a):
    nd = a.ndim
    return pl.BlockSpec(a.shape, lambda i, _n=nd: (0,) * _n)


@jax.jit
def _tc_node1(x, aggr, l1a, l1v, esc, b1c):
    return pl.pallas_call(
        _k1_body,
        grid=(NSTEP,),
        in_specs=[pl.BlockSpec((RB, D), lambda i: (i, 0)),
                  pl.BlockSpec((NC, RB, D), lambda i: (0, i, 0)),
                  _full(l1a), _full(l1v), _full(esc), _full(b1c)],
        out_specs=[pl.BlockSpec((RB, 64), lambda i: (i, 0)),
                   pl.BlockSpec((8, 64), lambda i: (0, 0))],
        out_shape=[jax.ShapeDtypeStruct((N, 64), jnp.float32),
                   jax.ShapeDtypeStruct((8, 64), jnp.float32)],
    )(x, aggr, l1a, l1v, esc, b1c)


@jax.jit
def _tc_node2(h1, st1, cs1, l2bd, gb1, b2c):
    return pl.pallas_call(
        _k2_body,
        grid=(NSTEP,),
        in_specs=[pl.BlockSpec((RB, 64), lambda i: (i, 0)),
                  pl.BlockSpec((8, 64), lambda i: (0, 0)),
                  pl.BlockSpec((8, 64), lambda i: (0, 0)),
                  _full(l2bd), _full(gb1), _full(b2c)],
        out_specs=[pl.BlockSpec((RB, 64), lambda i: (i, 0)),
                   pl.BlockSpec((8, 64), lambda i: (0, 0))],
        out_shape=[jax.ShapeDtypeStruct((N, 64), jnp.float32),
                   jax.ShapeDtypeStruct((8, 64), jnp.float32)],
    )(h1, st1, cs1, l2bd, gb1, b2c)


@jax.jit
def _tc_node3(h2, st2, cs2, gb2, obd, obc, w4, b4):
    return pl.pallas_call(
        _k3_body,
        grid=(NSTEP,),
        in_specs=[pl.BlockSpec((RB, 64), lambda i: (i, 0)),
                  pl.BlockSpec((8, 64), lambda i: (0, 0)),
                  pl.BlockSpec((8, 64), lambda i: (0, 0)),
                  _full(gb2), _full(obd), _full(obc), _full(w4), _full(b4)],
        out_specs=pl.BlockSpec((RB, 8), lambda i: (i, 0)),
        out_shape=jax.ShapeDtypeStruct((N, 8), jnp.float32),
    )(h2, st2, cs2, gb2, obd, obc, w4, b4)


def kernel(x, edge_index, edge_attr, batch, mask, params):
    src = edge_index[0].astype(jnp.int32)
    dst = edge_index[1].astype(jnp.int32)
    pad = EPAD - E
    src2d = jnp.concatenate([src, jnp.zeros((pad,), jnp.int32)]).reshape(-1, 128)
    dst2d = jnp.concatenate([dst, jnp.full((pad,), N, jnp.int32)]).reshape(-1, 128)
    ea_p = jnp.concatenate([edge_attr, jnp.zeros((pad, DE), jnp.float32)])

    pa, pv = params["actor_latent"], params["value_latent"]
    gx = _sc_gather(x, src2d)
    ew = jnp.stack([pa["edge_w"].T, pv["edge_w"].T])
    ebias = jnp.stack([pa["edge_b"][None, :], pv["edge_b"][None, :]])
    msg = _tc_edge(gx, ea_p, ew, ebias)
    zeros = jnp.zeros((AGG, 128), jnp.float32)
    aggr = _sc_scatter(msg, dst2d, zeros)

    cat = lambda a, b: jnp.concatenate([a, b])
    esc = jnp.stack([1.0 + pa["eps"], 1.0 + pv["eps"]]).reshape(1, 2)
    b1c = cat(pa["l1_b"], pv["l1_b"])[None, :]
    l2bd = jnp.zeros((64, 64), jnp.float32)
    l2bd = l2bd.at[0:32, 0:32].set(pa["l2_w"].T).at[32:64, 32:64].set(pv["l2_w"].T)
    gb1 = jnp.stack([cat(pa["bn_nn_g"], pv["bn_nn_g"]),
                     cat(pa["bn_nn_b"], pv["bn_nn_b"])])
    b2c = cat(pa["l2_b"], pv["l2_b"])[None, :]
    gb2 = jnp.stack([cat(pa["bn1_g"], pv["bn1_g"]),
                     cat(pa["bn1_b"], pv["bn1_b"])])
    obd = jnp.zeros((64, 128), jnp.float32)
    obd = obd.at[0:32, 0:64].set(pa["out_w"].T).at[32:64, 64:128].set(pv["out_w"].T)
    obc = cat(pa["out_b"], pv["out_b"])[None, :]
    aw, ab = params["actor_w"], params["actor_b"]
    w4 = jnp.zeros((128, 72), jnp.float32)
    w4 = w4.at[0:64, 0:54].set(aw.T)
    w4 = w4.at[64:128, 64:65].set(params["critic_w"].T)
    b4 = jnp.zeros((1, 72), jnp.float32)
    b4 = b4.at[0, 0:54].set(ab)
    b4 = b4.at[0, 64].set(params["critic_b"][0])

    h1, st1 = _tc_node1(x, aggr, pa["l1_w"].T, pv["l1_w"].T, esc, b1c)
    cs1 = _tc_csq(h1, st1)
    h2, st2 = _tc_node2(h1, st1, cs1, l2bd, gb1, b2c)
    cs2 = _tc_csq(h2, st2)
    out8 = _tc_node3(h2, st2, cs2, gb2, obd, obc, w4, b4)

    action_t = out8[:, 0:3].astype(jnp.int32)
    n_envs, mx = mask.shape
    pad_actions = jnp.where(mask[..., None], action_t.reshape(n_envs, mx, 3), 0)
    pad_lp = jnp.where(mask, out8[:, 3].reshape(n_envs, mx), 0.0)
    entropy = out8[:, 4]
    pad_v = jnp.where(mask, out8[:, 5].reshape(n_envs, mx), 0.0)
    return (pad_actions, pad_lp, entropy, pad_v)


# speedup vs baseline: 1.0802x; 1.0802x over previous
import jax
import jax.numpy as jnp
from jax import lax
from jax.experimental import pallas as pl
from jax.experimental.pallas import tpu as pltpu
from jax.experimental.pallas import tpu_sc as plsc

N = 10000
D = 128
DE = 16
E = 320000

NC, NS = 2, 16
NW = NC * NS
NCH = 80
EPAD = NW * NCH * 128
CPT = (EPAD // 128) // NS
AGG = 10112
ROWS_PER_TILE = AGG // NS

_mesh = plsc.VectorSubcoreMesh(core_axis_name="c", subcore_axis_name="s")


NBUF = 5


def _gather_body(x_hbm, src2d_hbm, gx_hbm, idx_v, b0, b1, b2, b3, b4, gsem, wsem):
    bufs = (b0, b1, b2, b3, b4)
    c = lax.axis_index("c")
    s = lax.axis_index("s")
    wid = s * NC + c
    row0 = wid * NCH
    pltpu.sync_copy(src2d_hbm.at[pl.ds(row0, NCH)], idx_v)
    for b in range(NBUF):
        pltpu.async_copy(x_hbm.at[idx_v.at[b]], bufs[b], gsem.at[b])

    def outer(k, carry):
        for b in range(NBUF):
            j = k * NBUF + b
            pltpu.make_async_copy(x_hbm.at[pl.ds(0, 128)],
                                  bufs[b], gsem.at[b]).wait()
            pltpu.async_copy(bufs[b],
                             gx_hbm.at[pl.ds((row0 + j) * 128, 128)], wsem.at[b])
            pltpu.make_async_copy(bufs[b],
                                  gx_hbm.at[pl.ds(0, 128)], wsem.at[b]).wait()

            @pl.when(j + NBUF < NCH)
            def _next():
                pltpu.async_copy(x_hbm.at[idx_v.at[j + NBUF]],
                                 bufs[b], gsem.at[b])
        return carry

    lax.fori_loop(0, NCH // NBUF, outer, 0)


@jax.jit
def _sc_gather(x, src2d):
    return pl.kernel(
        _gather_body,
        out_type=jax.ShapeDtypeStruct((EPAD, D), jnp.float32),
        mesh=_mesh,
        scratch_types=[
            pltpu.VMEM((NCH, 128), jnp.int32),
            pltpu.VMEM((128, D), jnp.float32),
            pltpu.VMEM((128, D), jnp.float32),
            pltpu.VMEM((128, D), jnp.float32),
            pltpu.VMEM((128, D), jnp.float32),
            pltpu.VMEM((128, D), jnp.float32),
            pltpu.SemaphoreType.DMA((NBUF,)),
            pltpu.SemaphoreType.DMA((NBUF,)),
        ],
    )(x, src2d)


SSEG = 80


def _scatter_body(msg_hbm, dst2d_hbm, zeros_hbm, out_hbm, idx_v, m0, m1, acc,
                  lsem):
    mbufs = (m0, m1)
    c = lax.axis_index("c")
    s = lax.axis_index("s")
    row0 = s * CPT

    @pl.when(s == 0)
    def _zero():
        pltpu.sync_copy(zeros_hbm, acc)

    plsc.subcore_barrier()
    for seg in range(CPT // SSEG):
        base = row0 + seg * SSEG
        pltpu.sync_copy(dst2d_hbm.at[pl.ds(base, SSEG)], idx_v)
        pltpu.async_copy(msg_hbm.at[c, pl.ds(base * 128, 128)], m0, lsem.at[0])

        def body(k, carry, base=base):
            for b in range(2):
                j = k * 2 + b
                pltpu.make_async_copy(msg_hbm.at[c, pl.ds(0, 128)],
                                      mbufs[b], lsem.at[b]).wait()

                @pl.when(j + 1 < SSEG)
                def _next():
                    pltpu.async_copy(msg_hbm.at[c, pl.ds((base + j + 1) * 128, 128)],
                                     mbufs[1 - b], lsem.at[1 - b])

                pltpu.sync_copy(mbufs[b], acc.at[idx_v.at[j]], add=True)
            return carry

        lax.fori_loop(0, SSEG // 2, body, 0)
    plsc.subcore_barrier()
    pltpu.sync_copy(acc.at[pl.ds(s * ROWS_PER_TILE, ROWS_PER_TILE)],
                    out_hbm.at[c, pl.ds(s * ROWS_PER_TILE, ROWS_PER_TILE)])


@jax.jit
def _sc_scatter(msg, dst2d, zeros):
    return pl.kernel(
        _scatter_body,
        out_type=jax.ShapeDtypeStruct((NC, AGG, 128), jnp.float32),
        mesh=_mesh,
        scratch_types=[
            pltpu.VMEM((SSEG, 128), jnp.int32),
            pltpu.VMEM((128, 128), jnp.float32),
            pltpu.VMEM((128, 128), jnp.float32),
            pltpu.VMEM_SHARED((AGG, 128), jnp.float32),
            pltpu.SemaphoreType.DMA((2,)),
        ],
    )(msg, dst2d, zeros)


EB = 4096


def _edge_mlp_body(gx_ref, ea_ref, ew_ref, ebias_ref, out_ref):
    gxt = gx_ref[...]
    ea = ea_ref[...]
    for c in range(NC):
        e = jnp.dot(ea, ew_ref[c], preferred_element_type=jnp.float32)
        out_ref[c] = jnp.maximum(gxt + e + ebias_ref[c], 0.0)


@jax.jit
def _tc_edge(gx, ea_p, ew, ebias):
    return pl.pallas_call(
        _edge_mlp_body,
        grid=(EPAD // EB,),
        in_specs=[
            pl.BlockSpec((EB, D), lambda i: (i, 0)),
            pl.BlockSpec((EB, DE), lambda i: (i, 0)),
            pl.BlockSpec((NC, DE, D), lambda i: (0, 0, 0)),
            pl.BlockSpec((NC, 1, D), lambda i: (0, 0, 0)),
        ],
        out_specs=pl.BlockSpec((NC, EB, D), lambda i: (0, i, 0)),
        out_shape=jax.ShapeDtypeStruct((NC, EPAD, D), jnp.float32),
    )(gx, ea_p, ew, ebias)


RB = 2000
NSTEP = N // RB


def _affine_gelu(h, st, cs, gb):
    m = st[0:1, :] * (1.0 / N)
    v = cs[0:1, :] * (1.0 / N)
    scale = gb[0:1, :] * lax.rsqrt(v + 1e-5)
    h = h * scale + (gb[1:2, :] - m * scale)
    return h * 0.5 * (1.0 + lax.erf(h * 0.7071067811865476))


def _csq_body(h_ref, st_ref, cs_ref):
    i = pl.program_id(0)
    m = st_ref[0:1, :] * (1.0 / N)
    d = h_ref[...] - m
    val = jnp.concatenate([jnp.sum(d * d, 0, keepdims=True),
                           jnp.zeros((7, d.shape[1]), jnp.float32)], 0)

    @pl.when(i == 0)
    def _init():
        cs_ref[...] = val

    @pl.when(i > 0)
    def _acc():
        cs_ref[...] = cs_ref[...] + val


@jax.jit
def _tc_csq(h, st):
    return pl.pallas_call(
        _csq_body,
        grid=(NSTEP,),
        in_specs=[pl.BlockSpec((RB, 64), lambda i: (i, 0)),
                  pl.BlockSpec((8, 64), lambda i: (0, 0))],
        out_specs=pl.BlockSpec((8, 64), lambda i: (0, 0)),
        out_shape=jax.ShapeDtypeStruct((8, 64), jnp.float32),
    )(h, st)


def _accum_stats(i, st_ref, h):
    s = jnp.sum(h, 0, keepdims=True)
    sq = jnp.sum(h * h, 0, keepdims=True)
    val = jnp.concatenate([s, sq, jnp.zeros((6, h.shape[1]), jnp.float32)], 0)

    @pl.when(i == 0)
    def _init():
        st_ref[...] = val

    @pl.when(i > 0)
    def _acc():
        st_ref[...] = st_ref[...] + val


def _k1_body(x_ref, agg_ref, l1a_ref, l1v_ref, esc_ref, b1_ref, h1_ref, st_ref):
    i = pl.program_id(0)
    x = x_ref[...]
    ha = jnp.dot(x * esc_ref[0, 0] + agg_ref[0],
                 l1a_ref[...], preferred_element_type=jnp.float32)
    hv = jnp.dot(x * esc_ref[0, 1] + agg_ref[1],
                 l1v_ref[...], preferred_element_type=jnp.float32)
    h = jnp.concatenate([ha, hv], axis=1) + b1_ref[...]
    h1_ref[...] = h
    _accum_stats(i, st_ref, h)


def _k2_body(h1_ref, st1_ref, cs1_ref, l2_ref, gb1_ref, b2_ref, h2_ref, st_ref):
    i = pl.program_id(0)
    g = _affine_gelu(h1_ref[...], st1_ref[...], cs1_ref[...], gb1_ref[...])
    h = jnp.dot(g, l2_ref[...], preferred_element_type=jnp.float32) + b2_ref[...]
    h2_ref[...] = h
    _accum_stats(i, st_ref, h)


def _k3_body(h2_ref, st2_ref, cs2_ref, gb2_ref, obd_ref, obc_ref, w4_ref, b4_ref,
             out_ref):
    g = _affine_gelu(h2_ref[...], st2_ref[...], cs2_ref[...], gb2_ref[...])
    ho = jnp.dot(g, obd_ref[...], preferred_element_type=jnp.float32) + obc_ref[...]
    r = jnp.dot(ho, w4_ref[...], preferred_element_type=jnp.float32) + b4_ref[...]
    lane = lax.broadcasted_iota(jnp.int32, r.shape, 1)
    lp_tot = jnp.zeros((RB, 1), jnp.float32)
    ent_tot = jnp.zeros((RB, 1), jnp.float32)
    ams = []
    for lo, hi in ((0, 36), (36, 46), (46, 54)):
        msk = (lane >= lo) & (lane < hi)
        lm = jnp.where(msk, r, -3e38)
        mx = jnp.max(lm, 1, keepdims=True)
        ex = jnp.where(msk, jnp.exp(r - mx), 0.0)
        s = jnp.sum(ex, 1, keepdims=True)
        lse = jnp.log(s) + mx
        am = jnp.min(jnp.where(lm == mx, lane, 10 ** 9), 1, keepdims=True)
        ams.append((am - lo).astype(jnp.float32))
        lp_tot = lp_tot + (mx - lse)
        ent_tot = ent_tot - jnp.sum(ex * (r - lse), 1, keepdims=True) / s
    val = r[:, 64:65]
    out_ref[...] = jnp.concatenate(
        ams + [lp_tot, ent_tot, val, jnp.zeros((RB, 2), jnp.float32)], axis=1)


def _full(a):
    nd = a.ndim
    return pl.BlockSpec(a.shape, lambda i, _n=nd: (0,) * _n)


@jax.jit
def _tc_node1(x, aggr, l1a, l1v, esc, b1c):
    return pl.pallas_call(
        _k1_body,
        grid=(NSTEP,),
        in_specs=[pl.BlockSpec((RB, D), lambda i: (i, 0)),
                  pl.BlockSpec((NC, RB, D), lambda i: (0, i, 0)),
                  _full(l1a), _full(l1v), _full(esc), _full(b1c)],
        out_specs=[pl.BlockSpec((RB, 64), lambda i: (i, 0)),
                   pl.BlockSpec((8, 64), lambda i: (0, 0))],
        out_shape=[jax.ShapeDtypeStruct((N, 64), jnp.float32),
                   jax.ShapeDtypeStruct((8, 64), jnp.float32)],
    )(x, aggr, l1a, l1v, esc, b1c)


@jax.jit
def _tc_node2(h1, st1, cs1, l2bd, gb1, b2c):
    return pl.pallas_call(
        _k2_body,
        grid=(NSTEP,),
        in_specs=[pl.BlockSpec((RB, 64), lambda i: (i, 0)),
                  pl.BlockSpec((8, 64), lambda i: (0, 0)),
                  pl.BlockSpec((8, 64), lambda i: (0, 0)),
                  _full(l2bd), _full(gb1), _full(b2c)],
        out_specs=[pl.BlockSpec((RB, 64), lambda i: (i, 0)),
                   pl.BlockSpec((8, 64), lambda i: (0, 0))],
        out_shape=[jax.ShapeDtypeStruct((N, 64), jnp.float32),
                   jax.ShapeDtypeStruct((8, 64), jnp.float32)],
    )(h1, st1, cs1, l2bd, gb1, b2c)


@jax.jit
def _tc_node3(h2, st2, cs2, gb2, obd, obc, w4, b4):
    return pl.pallas_call(
        _k3_body,
        grid=(NSTEP,),
        in_specs=[pl.BlockSpec((RB, 64), lambda i: (i, 0)),
                  pl.BlockSpec((8, 64), lambda i: (0, 0)),
                  pl.BlockSpec((8, 64), lambda i: (0, 0)),
                  _full(gb2), _full(obd), _full(obc), _full(w4), _full(b4)],
        out_specs=pl.BlockSpec((RB, 8), lambda i: (i, 0)),
        out_shape=jax.ShapeDtypeStruct((N, 8), jnp.float32),
    )(h2, st2, cs2, gb2, obd, obc, w4, b4)


def kernel(x, edge_index, edge_attr, batch, mask, params):
    src = edge_index[0].astype(jnp.int32)
    dst = edge_index[1].astype(jnp.int32)
    pad = EPAD - E
    src2d = jnp.concatenate([src, jnp.zeros((pad,), jnp.int32)]).reshape(-1, 128)
    dst2d = jnp.concatenate([dst, jnp.full((pad,), N, jnp.int32)]).reshape(-1, 128)
    ea_p = jnp.concatenate([edge_attr, jnp.zeros((pad, DE), jnp.float32)])

    pa, pv = params["actor_latent"], params["value_latent"]
    gx = _sc_gather(x, src2d)
    ew = jnp.stack([pa["edge_w"].T, pv["edge_w"].T])
    ebias = jnp.stack([pa["edge_b"][None, :], pv["edge_b"][None, :]])
    msg = _tc_edge(gx, ea_p, ew, ebias)
    zeros = jnp.zeros((AGG, 128), jnp.float32)
    aggr = _sc_scatter(msg, dst2d, zeros)

    cat = lambda a, b: jnp.concatenate([a, b])
    esc = jnp.stack([1.0 + pa["eps"], 1.0 + pv["eps"]]).reshape(1, 2)
    b1c = cat(pa["l1_b"], pv["l1_b"])[None, :]
    l2bd = jnp.zeros((64, 64), jnp.float32)
    l2bd = l2bd.at[0:32, 0:32].set(pa["l2_w"].T).at[32:64, 32:64].set(pv["l2_w"].T)
    gb1 = jnp.stack([cat(pa["bn_nn_g"], pv["bn_nn_g"]),
                     cat(pa["bn_nn_b"], pv["bn_nn_b"])])
    b2c = cat(pa["l2_b"], pv["l2_b"])[None, :]
    gb2 = jnp.stack([cat(pa["bn1_g"], pv["bn1_g"]),
                     cat(pa["bn1_b"], pv["bn1_b"])])
    obd = jnp.zeros((64, 128), jnp.float32)
    obd = obd.at[0:32, 0:64].set(pa["out_w"].T).at[32:64, 64:128].set(pv["out_w"].T)
    obc = cat(pa["out_b"], pv["out_b"])[None, :]
    aw, ab = params["actor_w"], params["actor_b"]
    w4 = jnp.zeros((128, 72), jnp.float32)
    w4 = w4.at[0:64, 0:54].set(aw.T)
    w4 = w4.at[64:128, 64:65].set(params["critic_w"].T)
    b4 = jnp.zeros((1, 72), jnp.float32)
    b4 = b4.at[0, 0:54].set(ab)
    b4 = b4.at[0, 64].set(params["critic_b"][0])

    h1, st1 = _tc_node1(x, aggr, pa["l1_w"].T, pv["l1_w"].T, esc, b1c)
    cs1 = _tc_csq(h1, st1)
    h2, st2 = _tc_node2(h1, st1, cs1, l2bd, gb1, b2c)
    cs2 = _tc_csq(h2, st2)
    out8 = _tc_node3(h2, st2, cs2, gb2, obd, obc, w4, b4)

    action_t = out8[:, 0:3].astype(jnp.int32)
    n_envs, mx = mask.shape
    pad_actions = jnp.where(mask[..., None], action_t.reshape(n_envs, mx, 3), 0)
    pad_lp = jnp.where(mask, out8[:, 3].reshape(n_envs, mx), 0.0)
    entropy = out8[:, 4]
    pad_v = jnp.where(mask, out8[:, 5].reshape(n_envs, mx), 0.0)
    return (pad_actions, pad_lp, entropy, pad_v)
